# baseline (speedup 1.0000x reference)
import functools

import jax
import jax.numpy as jnp
from jax import lax
from jax.experimental import pallas as pl
from jax.experimental.pallas import tpu as pltpu
from jax.experimental.pallas import tpu_sc as plsc

N = 10000
NE = 320000
D = 128
NP = 10240
NC = 2
NS = 16
NW = NC * NS
EPW = NE // NW
K = 80
BLK = 2000
NCHUNK = BLK // K
NPAIR = (NCHUNK - 1) // 2
NBLK = EPW // BLK
RPS = NP // NS
EPS = 1e-16


def _proj_body(e_ref, r_ref, w_ref, pq_ref, s_ref):
    w = w_ref[...]
    pq_ref[...] = jnp.dot(e_ref[...], w, preferred_element_type=jnp.float32)
    s_ref[...] = jnp.dot(r_ref[...], w, preferred_element_type=jnp.float32)


def _scalar_body(h_hbm, t_hbm, r_hbm, p_hbm, q_hbm, s_hbm,
                 ex_hbm, den_hbm,
                 h_v, t_v, r_v, p_v, q_v, s_v, den_v, ex_st, ld_sem):
    c = lax.axis_index("c")
    s_id = lax.axis_index("s")
    wid = s_id * NC + c
    eb = wid * EPW
    lsem = ld_sem
    pltpu.async_copy(h_hbm.at[pl.ds(eb, EPW)], h_v, lsem)
    pltpu.async_copy(t_hbm.at[pl.ds(eb, EPW)], t_v, lsem)
    pltpu.async_copy(r_hbm.at[pl.ds(eb, EPW)], r_v, lsem)
    pltpu.async_copy(p_hbm, p_v, lsem)
    pltpu.async_copy(q_hbm, q_v, lsem)
    pltpu.async_copy(s_hbm, s_v, lsem)
    pltpu.make_async_copy(h_hbm.at[pl.ds(eb, EPW)], h_v, lsem).wait()
    pltpu.make_async_copy(t_hbm.at[pl.ds(eb, EPW)], t_v, lsem).wait()
    pltpu.make_async_copy(r_hbm.at[pl.ds(eb, EPW)], r_v, lsem).wait()
    pltpu.make_async_copy(p_hbm, p_v, lsem).wait()
    pltpu.make_async_copy(q_hbm, q_v, lsem).wait()
    pltpu.make_async_copy(s_hbm, s_v, lsem).wait()

    z16 = jnp.zeros((16,), jnp.float32)

    def _zden(i, _):
        den_v[pl.ds(i * 16, 16)] = z16
        return _
    lax.fori_loop(0, NP // 16, _zden, None)

    def _edge16(g, _):
        off = g * 16
        ih = h_v[pl.ds(off, 16)]
        it = t_v[pl.ds(off, 16)]
        ir = r_v[pl.ds(off, 16)]
        pv = plsc.load_gather(p_v, [ih])
        qv = plsc.load_gather(q_v, [it])
        sv = plsc.load_gather(s_v, [ir])
        e = pv + qv + sv
        e = jnp.where(e > 0.0, e, e * 0.01)
        ex = jnp.exp(e)
        plsc.addupdate_scatter(den_v, [ih], ex)
        ex_st[pl.ds(off, 16)] = ex
        return _
    lax.fori_loop(0, EPW // 16, _edge16, None)

    pltpu.sync_copy(ex_st, ex_hbm.at[pl.ds(eb, EPW)])
    pltpu.sync_copy(den_v, den_hbm.at[wid])


_scalar_kernel = functools.partial(
    pl.kernel,
    out_type=[
        jax.ShapeDtypeStruct((NE,), jnp.float32),
        jax.ShapeDtypeStruct((NW, NP), jnp.float32),
    ],
    mesh=plsc.VectorSubcoreMesh(core_axis_name="c", subcore_axis_name="s"),
    compiler_params=pltpu.CompilerParams(needs_layout_passes=False),
    scratch_types=[
        pltpu.VMEM((EPW,), jnp.int32),
        pltpu.VMEM((EPW,), jnp.int32),
        pltpu.VMEM((EPW,), jnp.int32),
        pltpu.VMEM((N,), jnp.float32),
        pltpu.VMEM((N,), jnp.float32),
        pltpu.VMEM((N,), jnp.float32),
        pltpu.VMEM((NP,), jnp.float32),
        pltpu.VMEM((EPW,), jnp.float32),
        pltpu.SemaphoreType.DMA,
    ],
)(_scalar_body)


def _spmm_body(E_hbm, h_hbm, t_hbm, ex_hbm,
               outp_hbm,
               h_b, t_b, ex_b,
               rows0, rows1, rows2, rows3,
               ih0, ih1, ih2, ih3, it0, it1, it2, it3,
               sg0, sg1, sg2, sg3, ss0, ss1, ss2, ss3, accum):
    c = lax.axis_index("c")
    s_id = lax.axis_index("s")
    wid = s_id * NC + c
    eb = wid * EPW

    rows = [rows0, rows1, rows2, rows3]
    ihs = [ih0, ih1, ih2, ih3]
    its = [it0, it1, it2, it3]
    sgs = [sg0, sg1, sg2, sg3]
    sss = [ss0, ss1, ss2, ss3]

    z16 = jnp.zeros((16,), jnp.float32)

    def _zrows(k, _):
        for c8 in range(8):
            rows0[k, pl.ds(c8 * 16, 16)] = z16
        return _
    lax.fori_loop(0, K, _zrows, None)

    def _zacc(j, _):
        pltpu.sync_copy(rows0, accum.at[pl.ds(s_id * RPS + j * K, K)])
        return _
    lax.fori_loop(0, RPS // K, _zacc, None)
    plsc.subcore_barrier()

    def _build_idx(g, B):
        base = g * K
        for u in range(K // 16):
            sl = pl.ds(base + u * 16, 16)
            dl = pl.ds(u * 16, 16)
            ihs[B][dl] = h_b[sl]
            its[B][dl] = t_b[sl]

    def _fire_g(B):
        pltpu.async_copy(E_hbm.at[its[B]], rows[B], sgs[B])

    def _drain_g(B):
        pltpu.make_async_copy(E_hbm.at[its[B]], rows[B], sgs[B]).wait()

    def _fire_s(B):
        pltpu.async_copy(rows[B], accum.at[ihs[B]], sss[B], add=True)

    def _drain_s(B):
        pltpu.make_async_copy(rows[B], accum.at[ihs[B]], sss[B]).wait()

    def _scale(g, B):
        base = g * K
        r = rows[B]

        def _grp(j, _):
            for u in range(16):
                k = j * 16 + u
                exv = plsc.load_gather(ex_b, [jnp.full((16,), base + k,
                                                       jnp.int32)])
                for c8 in range(8):
                    sl2 = pl.ds(c8 * 16, 16)
                    r[k, sl2] = r[k, sl2] * exv
            return _
        lax.fori_loop(0, K // 16, _grp, None)

    def _proc(g, B, fire_ahead, drain_behind):
        if drain_behind:
            _drain_s((B + 2) % 4)
        if fire_ahead:
            _build_idx(g + 2, (B + 2) % 4)
            _fire_g((B + 2) % 4)
        _drain_g(B)
        _scale(g, B)
        _fire_s(B)

    def _block(b, _):
        boff = eb + b * BLK
        pltpu.sync_copy(h_hbm.at[pl.ds(boff, BLK)], h_b)
        pltpu.sync_copy(t_hbm.at[pl.ds(boff, BLK)], t_b)
        pltpu.sync_copy(ex_hbm.at[pl.ds(boff, BLK)], ex_b)

        _build_idx(0, 0)
        _fire_g(0)
        _build_idx(1, 1)
        _fire_g(1)
        _proc(0, 0, True, False)
        _proc(1, 1, True, False)

        def _quad(p_, _2):
            g0 = 4 * p_ + 2
            _proc(g0, 2, True, True)
            _proc(g0 + 1, 3, True, True)
            _proc(g0 + 2, 0, True, True)
            _proc(g0 + 3, 1, True, True)
            return _2
        lax.fori_loop(0, 5, _quad, None)

        _proc(22, 2, True, True)
        _proc(23, 3, False, True)
        _proc(24, 0, False, True)
        _drain_s(3)
        _drain_s(0)
        return _
    lax.fori_loop(0, NBLK, _block, None)

    plsc.subcore_barrier()

    def _dump(j, _):
        start = s_id * RPS + j * K
        pltpu.sync_copy(accum.at[pl.ds(start, K)], rows0)
        pltpu.sync_copy(rows0, outp_hbm.at[c, pl.ds(start, K)])
        return _
    lax.fori_loop(0, RPS // K, _dump, None)


_spmm_kernel = functools.partial(
    pl.kernel,
    out_type=jax.ShapeDtypeStruct((NC, NP, D), jnp.float32),
    mesh=plsc.VectorSubcoreMesh(core_axis_name="c", subcore_axis_name="s"),
    compiler_params=pltpu.CompilerParams(needs_layout_passes=False),
    scratch_types=[
        pltpu.VMEM((BLK,), jnp.int32),
        pltpu.VMEM((BLK,), jnp.int32),
        pltpu.VMEM((BLK,), jnp.float32),
        pltpu.VMEM((K, D), jnp.float32),
        pltpu.VMEM((K, D), jnp.float32),
        pltpu.VMEM((K, D), jnp.float32),
        pltpu.VMEM((K, D), jnp.float32),
        pltpu.VMEM((K,), jnp.int32),
        pltpu.VMEM((K,), jnp.int32),
        pltpu.VMEM((K,), jnp.int32),
        pltpu.VMEM((K,), jnp.int32),
        pltpu.VMEM((K,), jnp.int32),
        pltpu.VMEM((K,), jnp.int32),
        pltpu.VMEM((K,), jnp.int32),
        pltpu.VMEM((K,), jnp.int32),
        pltpu.SemaphoreType.DMA,
        pltpu.SemaphoreType.DMA,
        pltpu.SemaphoreType.DMA,
        pltpu.SemaphoreType.DMA,
        pltpu.SemaphoreType.DMA,
        pltpu.SemaphoreType.DMA,
        pltpu.SemaphoreType.DMA,
        pltpu.SemaphoreType.DMA,
        pltpu.VMEM_SHARED((NP, D), jnp.float32),
    ],
)(_spmm_body)


def _fin_body(p0_ref, p1_ref, den_ref, out_ref):
    d = jnp.sum(den_ref[...], axis=0)
    acc = p0_ref[...] + p1_ref[...]
    out_ref[...] = jnp.maximum(acc / (d[:, None] + EPS), 0.0)


def kernel(E, R, T, W_i, W_j, W_k):
    h = T[:, 0]
    r = T[:, 1]
    t = T[:, 2]
    wmat = jnp.zeros((D, 8), jnp.float32).at[:, 0].set(W_i).at[:, 1].set(W_j)
    pq_e, pq_r = pl.pallas_call(
        _proj_body,
        out_shape=[
            jax.ShapeDtypeStruct((N, 8), jnp.float32),
            jax.ShapeDtypeStruct((N, 8), jnp.float32),
        ],
    )(E, R, wmat)
    p = pq_e[:, 0]
    q = pq_e[:, 1]
    s = pq_r[:, 1]

    ex, den = _scalar_kernel(h, t, r, p, q, s)
    outp = _spmm_kernel(E, h, t, ex)

    blk = 1024
    e_new = pl.pallas_call(
        _fin_body,
        grid=(NP // blk,),
        in_specs=[
            pl.BlockSpec((blk, D), lambda i: (i, 0)),
            pl.BlockSpec((blk, D), lambda i: (i, 0)),
            pl.BlockSpec((NW, blk), lambda i: (0, i)),
        ],
        out_specs=pl.BlockSpec((blk, D), lambda i: (i, 0)),
        out_shape=jax.ShapeDtypeStruct((NP, D), jnp.float32),
    )(outp[0], outp[1], den)
    return (e_new[:N], R)


# speedup vs baseline: 1.0560x; 1.0560x over previous
import functools

import jax
import jax.numpy as jnp
from jax import lax
from jax.experimental import pallas as pl
from jax.experimental.pallas import tpu as pltpu
from jax.experimental.pallas import tpu_sc as plsc

N = 10000
NE = 320000
D = 128
NP = 10240
NC = 2
NS = 16
NW = NC * NS
EPW = NE // NW
K = 80
BLK = 2000
NCHUNK = BLK // K
NPAIR = (NCHUNK - 1) // 2
NBLK = EPW // BLK
RPS = NP // NS
EPS = 1e-16


def _proj_body(e_ref, r_ref, w_ref, pq_ref, s_ref):
    w = w_ref[...]
    pq_ref[...] = jnp.dot(e_ref[...], w, preferred_element_type=jnp.float32)
    s_ref[...] = jnp.dot(r_ref[...], w, preferred_element_type=jnp.float32)


def _scalar_body(h_hbm, t_hbm, r_hbm, p_hbm, q_hbm, s_hbm,
                 ex_hbm, den_hbm,
                 h_v, t_v, r_v, p_v, q_v, s_v, den_v, ex_st, ld_sem):
    c = lax.axis_index("c")
    s_id = lax.axis_index("s")
    wid = s_id * NC + c
    eb = wid * EPW
    lsem = ld_sem
    pltpu.async_copy(h_hbm.at[pl.ds(eb, EPW)], h_v, lsem)
    pltpu.async_copy(t_hbm.at[pl.ds(eb, EPW)], t_v, lsem)
    pltpu.async_copy(r_hbm.at[pl.ds(eb, EPW)], r_v, lsem)
    pltpu.async_copy(p_hbm, p_v, lsem)
    pltpu.async_copy(q_hbm, q_v, lsem)
    pltpu.async_copy(s_hbm, s_v, lsem)
    pltpu.make_async_copy(h_hbm.at[pl.ds(eb, EPW)], h_v, lsem).wait()
    pltpu.make_async_copy(t_hbm.at[pl.ds(eb, EPW)], t_v, lsem).wait()
    pltpu.make_async_copy(r_hbm.at[pl.ds(eb, EPW)], r_v, lsem).wait()
    pltpu.make_async_copy(p_hbm, p_v, lsem).wait()
    pltpu.make_async_copy(q_hbm, q_v, lsem).wait()
    pltpu.make_async_copy(s_hbm, s_v, lsem).wait()

    z16 = jnp.zeros((16,), jnp.float32)

    def _zden(i, _):
        den_v[pl.ds(i * 16, 16)] = z16
        return _
    lax.fori_loop(0, NP // 16, _zden, None)

    def _edge16(g, _):
        off = g * 16
        ih = h_v[pl.ds(off, 16)]
        it = t_v[pl.ds(off, 16)]
        ir = r_v[pl.ds(off, 16)]
        pv = plsc.load_gather(p_v, [ih])
        qv = plsc.load_gather(q_v, [it])
        sv = plsc.load_gather(s_v, [ir])
        e = pv + qv + sv
        e = jnp.where(e > 0.0, e, e * 0.01)
        ex = jnp.exp(e)
        plsc.addupdate_scatter(den_v, [ih], ex)
        ex_st[pl.ds(off, 16)] = ex
        return _
    lax.fori_loop(0, EPW // 16, _edge16, None)

    pltpu.sync_copy(ex_st, ex_hbm.at[pl.ds(eb, EPW)])
    pltpu.sync_copy(den_v, den_hbm.at[wid])


_scalar_kernel = functools.partial(
    pl.kernel,
    out_type=[
        jax.ShapeDtypeStruct((NE,), jnp.float32),
        jax.ShapeDtypeStruct((NW, NP), jnp.float32),
    ],
    mesh=plsc.VectorSubcoreMesh(core_axis_name="c", subcore_axis_name="s"),
    compiler_params=pltpu.CompilerParams(needs_layout_passes=False),
    scratch_types=[
        pltpu.VMEM((EPW,), jnp.int32),
        pltpu.VMEM((EPW,), jnp.int32),
        pltpu.VMEM((EPW,), jnp.int32),
        pltpu.VMEM((N,), jnp.float32),
        pltpu.VMEM((N,), jnp.float32),
        pltpu.VMEM((N,), jnp.float32),
        pltpu.VMEM((NP,), jnp.float32),
        pltpu.VMEM((EPW,), jnp.float32),
        pltpu.SemaphoreType.DMA,
    ],
)(_scalar_body)


def _spmm_body(E_hbm, h_hbm, t_hbm, ex_hbm,
               outp_hbm,
               h_b, t_b, ex_b,
               rows0, rows1, rows2, rows3,
               ih0, ih1, ih2, ih3, it0, it1, it2, it3,
               sg0, sg1, sg2, sg3, ss0, ss1, ss2, ss3, st_sem, accum):
    c = lax.axis_index("c")
    s_id = lax.axis_index("s")
    wid = s_id * NC + c
    eb = wid * EPW

    rows = [rows0, rows1, rows2, rows3]
    ihs = [ih0, ih1, ih2, ih3]
    its = [it0, it1, it2, it3]
    sgs = [sg0, sg1, sg2, sg3]
    sss = [ss0, ss1, ss2, ss3]

    z16 = jnp.zeros((16,), jnp.float32)

    def _zrows(k, _):
        for c8 in range(8):
            rows0[k, pl.ds(c8 * 16, 16)] = z16
        return _
    lax.fori_loop(0, K, _zrows, None)

    def _zacc(j, _):
        pltpu.sync_copy(rows0, accum.at[pl.ds(s_id * RPS + j * K, K)])
        return _
    lax.fori_loop(0, RPS // K, _zacc, None)
    plsc.subcore_barrier()

    def _build_idx(g, B):
        base = g * K
        for u in range(K // 16):
            sl = pl.ds(base + u * 16, 16)
            dl = pl.ds(u * 16, 16)
            ihs[B][dl] = h_b[sl]
            its[B][dl] = t_b[sl]

    def _fire_g(B):
        pltpu.async_copy(E_hbm.at[its[B]], rows[B], sgs[B])

    def _drain_g(B):
        pltpu.make_async_copy(E_hbm.at[its[B]], rows[B], sgs[B]).wait()

    def _fire_s(B):
        pltpu.async_copy(rows[B], accum.at[ihs[B]], sss[B], add=True)

    def _drain_s(B):
        pltpu.make_async_copy(rows[B], accum.at[ihs[B]], sss[B]).wait()

    def _scale(g, B):
        base = g * K
        r = rows[B]

        def _grp(j, _):
            ex16 = ex_b[pl.ds(base + j * 16, 16)]
            for u in range(16):
                sc = ex16[u]
                k = j * 16 + u
                for c8 in range(8):
                    sl2 = pl.ds(c8 * 16, 16)
                    r[k, sl2] = r[k, sl2] * sc
            return _
        lax.fori_loop(0, K // 16, _grp, None)

    def _proc(g, B, fire_ahead, drain_behind):
        if drain_behind:
            _drain_s((B + 2) % 4)
        if fire_ahead:
            _build_idx(g + 2, (B + 2) % 4)
            _fire_g((B + 2) % 4)
        _drain_g(B)
        _scale(g, B)
        _fire_s(B)

    def _stage_ht_fire(b):
        boff = eb + b * BLK
        pltpu.async_copy(h_hbm.at[pl.ds(boff, BLK)], h_b, st_sem)
        pltpu.async_copy(t_hbm.at[pl.ds(boff, BLK)], t_b, st_sem)

    def _stage_ht_drain(b):
        boff = eb + b * BLK
        pltpu.make_async_copy(h_hbm.at[pl.ds(boff, BLK)], h_b, st_sem).wait()
        pltpu.make_async_copy(t_hbm.at[pl.ds(boff, BLK)], t_b, st_sem).wait()

    _stage_ht_fire(0)

    def _block(b, _):
        boff = eb + b * BLK
        _stage_ht_drain(b)
        pltpu.sync_copy(ex_hbm.at[pl.ds(boff, BLK)], ex_b)

        _build_idx(0, 0)
        _fire_g(0)
        _build_idx(1, 1)
        _fire_g(1)
        _proc(0, 0, True, False)
        _proc(1, 1, True, False)

        def _quad(p_, _2):
            g0 = 4 * p_ + 2
            _proc(g0, 2, True, True)
            _proc(g0 + 1, 3, True, True)
            _proc(g0 + 2, 0, True, True)
            _proc(g0 + 3, 1, True, True)
            return _2
        lax.fori_loop(0, 5, _quad, None)

        _proc(22, 2, True, True)

        @pl.when(b + 1 < NBLK)
        def _prefetch():
            _stage_ht_fire(b + 1)

        _proc(23, 3, False, True)
        _proc(24, 0, False, True)
        _drain_s(3)
        _drain_s(0)
        return _
    lax.fori_loop(0, NBLK, _block, None)

    plsc.subcore_barrier()

    def _dump(j, _):
        start = s_id * RPS + j * K
        pltpu.sync_copy(accum.at[pl.ds(start, K)], rows0)
        pltpu.sync_copy(rows0, outp_hbm.at[c, pl.ds(start, K)])
        return _
    lax.fori_loop(0, RPS // K, _dump, None)


_spmm_kernel = functools.partial(
    pl.kernel,
    out_type=jax.ShapeDtypeStruct((NC, NP, D), jnp.float32),
    mesh=plsc.VectorSubcoreMesh(core_axis_name="c", subcore_axis_name="s"),
    compiler_params=pltpu.CompilerParams(needs_layout_passes=False),
    scratch_types=[
        pltpu.VMEM((BLK,), jnp.int32),
        pltpu.VMEM((BLK,), jnp.int32),
        pltpu.VMEM((BLK,), jnp.float32),
        pltpu.VMEM((K, D), jnp.float32),
        pltpu.VMEM((K, D), jnp.float32),
        pltpu.VMEM((K, D), jnp.float32),
        pltpu.VMEM((K, D), jnp.float32),
        pltpu.VMEM((K,), jnp.int32),
        pltpu.VMEM((K,), jnp.int32),
        pltpu.VMEM((K,), jnp.int32),
        pltpu.VMEM((K,), jnp.int32),
        pltpu.VMEM((K,), jnp.int32),
        pltpu.VMEM((K,), jnp.int32),
        pltpu.VMEM((K,), jnp.int32),
        pltpu.VMEM((K,), jnp.int32),
        pltpu.SemaphoreType.DMA,
        pltpu.SemaphoreType.DMA,
        pltpu.SemaphoreType.DMA,
        pltpu.SemaphoreType.DMA,
        pltpu.SemaphoreType.DMA,
        pltpu.SemaphoreType.DMA,
        pltpu.SemaphoreType.DMA,
        pltpu.SemaphoreType.DMA,
        pltpu.SemaphoreType.DMA,
        pltpu.VMEM_SHARED((NP, D), jnp.float32),
    ],
)(_spmm_body)


def _fin_body(p0_ref, p1_ref, den_ref, out_ref):
    d = jnp.sum(den_ref[...], axis=0)
    acc = p0_ref[...] + p1_ref[...]
    out_ref[...] = jnp.maximum(acc / (d[:, None] + EPS), 0.0)


def kernel(E, R, T, W_i, W_j, W_k):
    h = T[:, 0]
    r = T[:, 1]
    t = T[:, 2]
    wmat = jnp.zeros((D, 8), jnp.float32).at[:, 0].set(W_i).at[:, 1].set(W_j)
    pq_e, pq_r = pl.pallas_call(
        _proj_body,
        out_shape=[
            jax.ShapeDtypeStruct((N, 8), jnp.float32),
            jax.ShapeDtypeStruct((N, 8), jnp.float32),
        ],
    )(E, R, wmat)
    p = pq_e[:, 0]
    q = pq_e[:, 1]
    s = pq_r[:, 1]

    ex, den = _scalar_kernel(h, t, r, p, q, s)
    outp = _spmm_kernel(E, h, t, ex)

    blk = 1024
    e_new = pl.pallas_call(
        _fin_body,
        grid=(NP // blk,),
        in_specs=[
            pl.BlockSpec((blk, D), lambda i: (i, 0)),
            pl.BlockSpec((blk, D), lambda i: (i, 0)),
            pl.BlockSpec((NW, blk), lambda i: (0, i)),
        ],
        out_specs=pl.BlockSpec((blk, D), lambda i: (i, 0)),
        out_shape=jax.ShapeDtypeStruct((NP, D), jnp.float32),
    )(outp[0], outp[1], den)
    return (e_new[:N], R)
